# baseline (speedup 1.0000x reference)
import jax
import jax.numpy as jnp
from jax import lax
from jax.experimental import pallas as pl
from jax.experimental.pallas import tpu as pltpu
from jax.experimental.pallas import tpu_sc as plsc

B = 4
C = 64
P = 12000
H = 512
W = 512

NC = 2
NS = 16
L = 16

CHUNKS_PER_BATCH = (NC * NS) // B
ROWS_PER_CHUNK = H // CHUNKS_PER_BATCH
CHUNK_PIX = ROWS_PER_CHUNK * W

PGROUPS = P // L
MGROUPS = CHUNK_PIX // L

F_OFF = 0
XI_OFF = P
YI_OFF = 2 * P
XC0 = 0
XC1 = P
POOL = 3 * P

MAP_ROW = 0
STG0_ROW = 0
STG1_ROW = 64
LIST_ROW = 128
SH_ROW = 152
GRID_ROWS = 153

INT_MAX = 2**31 - 1
PBITS = 14


def _i(v):
  return plsc.bitcast(v, jnp.int32)


def _f(v):
  return plsc.bitcast(v, jnp.float32)


def _body(x_hbm, ind_hbm, out_hbm, pool, grid, sx0, sx1, ss0, ss1):
  cid = lax.axis_index("c")
  sid = lax.axis_index("s")
  wid = cid * NS + sid
  b = wid // CHUNKS_PER_BATCH
  chunk = wid % CHUNKS_PER_BATCH
  lo = chunk * CHUNK_PIX
  r0 = chunk * ROWS_PER_CHUNK
  iota = lax.iota(jnp.int32, L)
  fzeros = jnp.zeros((L,), jnp.float32)
  row_sh = jnp.full((L,), SH_ROW, jnp.int32)

  pltpu.async_copy(ind_hbm.at[pl.ds(b * P, P)], pool.at[pl.ds(F_OFF, P)], ss0)
  pltpu.async_copy(
      ind_hbm.at[pl.ds((B + b) * P, P)], pool.at[pl.ds(XI_OFF, P)], ss0)
  pltpu.async_copy(
      ind_hbm.at[pl.ds((2 * B + b) * P, P)], pool.at[pl.ds(YI_OFF, P)], ss0)

  def zero_map(i, _):
    for k in range(4):
      g = i * 4 + k
      grid[g >> 5, pl.ds((g & 31) * L, L)] = fzeros
    return _
  lax.fori_loop(0, 2 * MGROUPS // 4, zero_map, None)

  for off in (F_OFF, XI_OFF, YI_OFF):
    pltpu.make_async_copy(
        ind_hbm.at[pl.ds(b * P, P)], pool.at[pl.ds(off, P)], ss0).wait()

  def point_group(g, _):
    base = g * L
    f = _i(pool[pl.ds(F_OFF + base, L)])
    xv = _i(pool[pl.ds(XI_OFF + base, L)])
    yv = _i(pool[pl.ds(YI_OFF + base, L)])
    local = yv * W + xv - lo
    valid = (f != 0) & (local >= 0) & (local < CHUNK_PIX)
    key = jnp.where(valid, local * L + iota, INT_MAX)
    skey, sp = plsc.sort_key_val(key, base + iota)
    spix = lax.shift_right_arithmetic(skey, 4)
    grid[SH_ROW, pl.ds(0, L)] = _f(jnp.full((L,), -1, jnp.int32))
    plsc.store_scatter(
        grid, [row_sh, jnp.maximum(iota - 1, 0)], _f(spix), mask=iota > 0)
    npix = _i(grid[SH_ROW, pl.ds(0, L)])
    winner = (spix != npix) & (skey != INT_MAX)
    spix_c = spix & (CHUNK_PIX - 1)
    plsc.store_scatter(
        grid, [spix_c >> 9, spix_c & (W - 1)], _f(sp + 1), mask=winner)
    return _
  lax.fori_loop(0, PGROUPS, point_group, None)

  xbase = b * (C * P)
  pltpu.async_copy(x_hbm.at[pl.ds(xbase, P)], pool.at[pl.ds(XC0, P)], sx0)
  pltpu.async_copy(x_hbm.at[pl.ds(xbase + P, P)], pool.at[pl.ds(XC1, P)], sx1)

  def compact(mg, cnt):
    m = _i(grid[mg >> 5, pl.ds((mg & 31) * L, L)])
    msk = m > 0
    mi = msk.astype(jnp.int32)
    n = jnp.sum(mi)

    def emit(c):
      pos = jnp.maximum(c + plsc.cumsum(mi) - 1, 0)
      entry = ((mg * L + iota) << PBITS) + m - 1
      plsc.store_scatter(
          grid, [LIST_ROW + (pos >> 9), pos & (W - 1)], _f(entry), mask=msk)
      return c + n

    return lax.cond(n > 0, emit, lambda c: c, cnt)
  cnt = lax.fori_loop(0, MGROUPS, compact, jnp.int32(0))

  pad = cnt + iota
  plsc.store_scatter(
      grid, [LIST_ROW + (pad >> 9), pad & (W - 1)],
      _f(jnp.full((L,), -1, jnp.int32)))
  ngroups = (cnt + (L - 1)) // L

  def out_dst(c):
    return out_hbm.at[b, c, pl.ds(r0, ROWS_PER_CHUNK)]

  def do_channel(c, xc, srow, sem_x, sem_s):
    pltpu.make_async_copy(
        x_hbm.at[pl.ds(xbase + c * P, P)], pool.at[pl.ds(xc, P)], sem_x).wait()

    @pl.when(c >= 2)
    def _wait_stage():
      pltpu.make_async_copy(
          grid.at[pl.ds(srow, ROWS_PER_CHUNK)], out_dst(c - 2), sem_s).wait()

    def gather_group(g, _):
      e = _i(grid[LIST_ROW + (g >> 5), pl.ds((g & 31) * L, L)])
      live = e >= 0
      pv = e & ((1 << PBITS) - 1)
      li = (e >> PBITS) & (CHUNK_PIX - 1)
      vals = plsc.load_gather(pool, [xc + pv], mask=live)
      plsc.store_scatter(
          grid, [srow + (li >> 9), li & (W - 1)], vals, mask=live)
      return _
    lax.fori_loop(0, ngroups, gather_group, None)

    @pl.when(c < C - 2)
    def _prefetch():
      pltpu.async_copy(
          x_hbm.at[pl.ds(xbase + (c + 2) * P, P)], pool.at[pl.ds(xc, P)],
          sem_x)

    pltpu.async_copy(grid.at[pl.ds(srow, ROWS_PER_CHUNK)], out_dst(c), sem_s)

  def chan_pair(cc, _):
    do_channel(cc * 2, XC0, STG0_ROW, sx0, ss0)
    do_channel(cc * 2 + 1, XC1, STG1_ROW, sx1, ss1)
    return _
  lax.fori_loop(0, C // 2, chan_pair, None)

  pltpu.make_async_copy(
      grid.at[pl.ds(STG0_ROW, ROWS_PER_CHUNK)], out_dst(C - 2), ss0).wait()
  pltpu.make_async_copy(
      grid.at[pl.ds(STG1_ROW, ROWS_PER_CHUNK)], out_dst(C - 1), ss1).wait()


_mesh = plsc.VectorSubcoreMesh(
    core_axis_name="c", subcore_axis_name="s", num_cores=NC, num_subcores=NS)

_sc_scatter = pl.kernel(
    _body,
    out_type=jax.ShapeDtypeStruct((B, C, H, W), jnp.float32),
    mesh=_mesh,
    scratch_types=[
        pltpu.VMEM((POOL,), jnp.float32),
        pltpu.VMEM((GRID_ROWS, W), jnp.float32),
        pltpu.SemaphoreType.DMA,
        pltpu.SemaphoreType.DMA,
        pltpu.SemaphoreType.DMA,
        pltpu.SemaphoreType.DMA,
    ],
    compiler_params=pltpu.CompilerParams(needs_layout_passes=False),
)


@jax.jit
def kernel(x, inds):
  ind_t = lax.bitcast_convert_type(
      jnp.transpose(inds, (2, 0, 1)).reshape(-1), jnp.float32)
  return _sc_scatter(x.reshape(-1), ind_t)


# speedup vs baseline: 1.1099x; 1.1099x over previous
import jax
import jax.numpy as jnp
from jax import lax
from jax.experimental import pallas as pl
from jax.experimental.pallas import tpu as pltpu
from jax.experimental.pallas import tpu_sc as plsc

B = 4
C = 64
P = 12000
H = 512
W = 512

NC = 2
NS = 16
L = 16

CHUNKS_PER_BATCH = (NC * NS) // B
ROWS_PER_CHUNK = H // CHUNKS_PER_BATCH
CHUNK_PIX = ROWS_PER_CHUNK * W

PGROUPS = P // L
MGROUPS = CHUNK_PIX // L

F_OFF = 0
XI_OFF = P
YI_OFF = 2 * P
XC0 = 0
XC1 = P
POOL = 3 * P

MAP_ROW = 0
STG0_ROW = 0
STG1_ROW = 64
LIST_ROW = 128
SH_ROW = 152
GRID_ROWS = 153

INT_MAX = 2**31 - 1
PBITS = 14


def _i(v):
  return plsc.bitcast(v, jnp.int32)


def _f(v):
  return plsc.bitcast(v, jnp.float32)


def _body(x_hbm, ind_hbm, out_hbm, pool, grid, sx0, sx1, ss0, ss1):
  cid = lax.axis_index("c")
  sid = lax.axis_index("s")
  wid = cid * NS + sid
  b = wid // CHUNKS_PER_BATCH
  chunk = wid % CHUNKS_PER_BATCH
  lo = chunk * CHUNK_PIX
  r0 = chunk * ROWS_PER_CHUNK
  iota = lax.iota(jnp.int32, L)
  fzeros = jnp.zeros((L,), jnp.float32)
  row_sh = jnp.full((L,), SH_ROW, jnp.int32)

  pltpu.async_copy(ind_hbm.at[pl.ds(b * P, P)], pool.at[pl.ds(F_OFF, P)], ss0)
  pltpu.async_copy(
      ind_hbm.at[pl.ds((B + b) * P, P)], pool.at[pl.ds(XI_OFF, P)], ss0)
  pltpu.async_copy(
      ind_hbm.at[pl.ds((2 * B + b) * P, P)], pool.at[pl.ds(YI_OFF, P)], ss0)

  def zero_map(i, _):
    for k in range(4):
      g = i * 4 + k
      grid[g >> 5, pl.ds((g & 31) * L, L)] = fzeros
    return _
  lax.fori_loop(0, 2 * MGROUPS // 4, zero_map, None)

  for off in (F_OFF, XI_OFF, YI_OFF):
    pltpu.make_async_copy(
        ind_hbm.at[pl.ds(b * P, P)], pool.at[pl.ds(off, P)], ss0).wait()

  def point_group(g, _):
    base = g * L
    f = _i(pool[pl.ds(F_OFF + base, L)])
    xv = _i(pool[pl.ds(XI_OFF + base, L)])
    yv = _i(pool[pl.ds(YI_OFF + base, L)])
    local = yv * W + xv - lo
    valid = (f != 0) & (local >= 0) & (local < CHUNK_PIX)
    key = jnp.where(valid, local * L + iota, INT_MAX)
    skey, sp = plsc.sort_key_val(key, base + iota)
    spix = lax.shift_right_arithmetic(skey, 4)
    grid[SH_ROW, pl.ds(0, L)] = _f(jnp.full((L,), -1, jnp.int32))
    plsc.store_scatter(
        grid, [row_sh, jnp.maximum(iota - 1, 0)], _f(spix), mask=iota > 0)
    npix = _i(grid[SH_ROW, pl.ds(0, L)])
    winner = (spix != npix) & (skey != INT_MAX)
    spix_c = spix & (CHUNK_PIX - 1)
    plsc.store_scatter(
        grid, [spix_c >> 9, spix_c & (W - 1)], _f(sp + 1), mask=winner)
    return _
  lax.fori_loop(0, PGROUPS, point_group, None)

  xbase = b * (C * P)
  pltpu.async_copy(x_hbm.at[pl.ds(xbase, P)], pool.at[pl.ds(XC0, P)], sx0)
  pltpu.async_copy(x_hbm.at[pl.ds(xbase + P, P)], pool.at[pl.ds(XC1, P)], sx1)

  def compact(mg, cnt):
    m = _i(grid[mg >> 5, pl.ds((mg & 31) * L, L)])
    msk = m > 0
    mi = msk.astype(jnp.int32)
    pos = jnp.maximum(cnt + plsc.cumsum(mi) - 1, 0)
    entry = ((mg * L + iota) << PBITS) + m - 1
    plsc.store_scatter(
        grid, [LIST_ROW + (pos >> 9), pos & (W - 1)], _f(entry), mask=msk)
    return cnt + jnp.sum(mi)
  cnt = lax.fori_loop(0, MGROUPS, compact, jnp.int32(0))

  pad = cnt + iota
  plsc.store_scatter(
      grid, [LIST_ROW + (pad >> 9), pad & (W - 1)],
      _f(jnp.full((L,), -1, jnp.int32)))
  ngroups = (cnt + (L - 1)) // L

  def out_dst(c):
    return out_hbm.at[b, c, pl.ds(r0, ROWS_PER_CHUNK)]

  def do_channel(c, xc, srow, sem_x, sem_s):
    pltpu.make_async_copy(
        x_hbm.at[pl.ds(xbase + c * P, P)], pool.at[pl.ds(xc, P)], sem_x).wait()

    @pl.when(c >= 2)
    def _wait_stage():
      pltpu.make_async_copy(
          grid.at[pl.ds(srow, ROWS_PER_CHUNK)], out_dst(c - 2), sem_s).wait()

    def gather_group(g, _):
      e = _i(grid[LIST_ROW + (g >> 5), pl.ds((g & 31) * L, L)])
      live = e >= 0
      pv = e & ((1 << PBITS) - 1)
      li = (e >> PBITS) & (CHUNK_PIX - 1)
      vals = plsc.load_gather(pool, [xc + pv], mask=live)
      plsc.store_scatter(
          grid, [srow + (li >> 9), li & (W - 1)], vals, mask=live)
      return _
    lax.fori_loop(0, ngroups, gather_group, None)

    @pl.when(c < C - 2)
    def _prefetch():
      pltpu.async_copy(
          x_hbm.at[pl.ds(xbase + (c + 2) * P, P)], pool.at[pl.ds(xc, P)],
          sem_x)

    pltpu.async_copy(grid.at[pl.ds(srow, ROWS_PER_CHUNK)], out_dst(c), sem_s)

  def chan_pair(cc, _):
    do_channel(cc * 2, XC0, STG0_ROW, sx0, ss0)
    do_channel(cc * 2 + 1, XC1, STG1_ROW, sx1, ss1)
    return _
  lax.fori_loop(0, C // 2, chan_pair, None)

  pltpu.make_async_copy(
      grid.at[pl.ds(STG0_ROW, ROWS_PER_CHUNK)], out_dst(C - 2), ss0).wait()
  pltpu.make_async_copy(
      grid.at[pl.ds(STG1_ROW, ROWS_PER_CHUNK)], out_dst(C - 1), ss1).wait()


_mesh = plsc.VectorSubcoreMesh(
    core_axis_name="c", subcore_axis_name="s", num_cores=NC, num_subcores=NS)

_sc_scatter = pl.kernel(
    _body,
    out_type=jax.ShapeDtypeStruct((B, C, H, W), jnp.float32),
    mesh=_mesh,
    scratch_types=[
        pltpu.VMEM((POOL,), jnp.float32),
        pltpu.VMEM((GRID_ROWS, W), jnp.float32),
        pltpu.SemaphoreType.DMA,
        pltpu.SemaphoreType.DMA,
        pltpu.SemaphoreType.DMA,
        pltpu.SemaphoreType.DMA,
    ],
    compiler_params=pltpu.CompilerParams(needs_layout_passes=False),
)


@jax.jit
def kernel(x, inds):
  ind_t = lax.bitcast_convert_type(
      jnp.transpose(inds, (2, 0, 1)).reshape(-1), jnp.float32)
  return _sc_scatter(x.reshape(-1), ind_t)


# speedup vs baseline: 1.2009x; 1.0820x over previous
import jax
import jax.numpy as jnp
from jax import lax
from jax.experimental import pallas as pl
from jax.experimental.pallas import tpu as pltpu
from jax.experimental.pallas import tpu_sc as plsc

B = 4
C = 64
P = 12000
H = 512
W = 512

NC = 2
NS = 16
L = 16

CHUNKS_PER_BATCH = (NC * NS) // B
ROWS_PER_CHUNK = H // CHUNKS_PER_BATCH
CHUNK_PIX = ROWS_PER_CHUNK * W
CPB = C // CHUNKS_PER_BATCH

PGROUPS = P // L
MGROUPS = CHUNK_PIX // L

F_OFF = 0
XI_OFF = P
YI_OFF = 2 * P
XC0 = 0
XC1 = P
POOL = 3 * P

STG0_ROW = 0
STG1_ROW = 64
LIST_ROW = 128
SH_ROW = 160
GRID_ROWS = 161

LIST_PUB_ROWS = 24

INT_MAX = 2**31 - 1
PBITS = 14


def _i(v):
  return plsc.bitcast(v, jnp.int32)


def _f(v):
  return plsc.bitcast(v, jnp.float32)


def _body(x_hbm, ind_hbm, out_hbm, pool, grid, xls, hdr,
          sx0, sx1, ss0, ss1, sf):
  cid = lax.axis_index("c")
  sid = lax.axis_index("s")
  bic = sid // CHUNKS_PER_BATCH
  idx8 = sid % CHUNKS_PER_BATCH
  b = cid * 2 + bic
  chunk = idx8
  lo = chunk * CHUNK_PIX
  iota = lax.iota(jnp.int32, L)
  fzeros = jnp.zeros((L,), jnp.float32)
  row_sh = jnp.full((L,), SH_ROW, jnp.int32)

  pltpu.async_copy(ind_hbm.at[pl.ds(b * P, P)], pool.at[pl.ds(F_OFF, P)], ss0)
  pltpu.async_copy(
      ind_hbm.at[pl.ds((B + b) * P, P)], pool.at[pl.ds(XI_OFF, P)], ss0)
  pltpu.async_copy(
      ind_hbm.at[pl.ds((2 * B + b) * P, P)], pool.at[pl.ds(YI_OFF, P)], ss0)

  def zero_all(i, _):
    for k in range(4):
      g = i * 4 + k
      grid[g >> 5, pl.ds((g & 31) * L, L)] = fzeros
    return _
  lax.fori_loop(0, 2 * MGROUPS // 4, zero_all, None)

  for off in (F_OFF, XI_OFF, YI_OFF):
    pltpu.make_async_copy(
        ind_hbm.at[pl.ds(b * P, P)], pool.at[pl.ds(off, P)], ss0).wait()

  def point_group(g, _):
    base = g * L
    f = _i(pool[pl.ds(F_OFF + base, L)])
    xv = _i(pool[pl.ds(XI_OFF + base, L)])
    yv = _i(pool[pl.ds(YI_OFF + base, L)])
    local = yv * W + xv - lo
    valid = (f != 0) & (local >= 0) & (local < CHUNK_PIX)
    key = jnp.where(valid, local * L + iota, INT_MAX)
    skey, sp = plsc.sort_key_val(key, base + iota)
    spix = lax.shift_right_arithmetic(skey, 4)
    grid[SH_ROW, pl.ds(0, L)] = _f(jnp.full((L,), -1, jnp.int32))
    plsc.store_scatter(
        grid, [row_sh, jnp.maximum(iota - 1, 0)], _f(spix), mask=iota > 0)
    npix = _i(grid[SH_ROW, pl.ds(0, L)])
    winner = (spix != npix) & (skey != INT_MAX)
    spix_c = spix & (CHUNK_PIX - 1)
    plsc.store_scatter(
        grid, [spix_c >> 9, spix_c & (W - 1)], _f(sp + 1), mask=winner)
    return _
  lax.fori_loop(0, PGROUPS, point_group, None)

  c0 = idx8 * CPB
  pltpu.async_copy(
      x_hbm.at[pl.ds((b * C + c0) * P, P)], pool.at[pl.ds(XC0, P)], sx0)
  pltpu.async_copy(
      x_hbm.at[pl.ds((b * C + c0 + 1) * P, P)], pool.at[pl.ds(XC1, P)], sx1)

  def compact(mg, cnt):
    m = _i(grid[mg >> 5, pl.ds((mg & 31) * L, L)])
    msk = m > 0
    mi = msk.astype(jnp.int32)
    pos = jnp.maximum(cnt + plsc.cumsum(mi) - 1, 0)
    entry = ((mg * L + iota) << PBITS) + m - 1
    plsc.store_scatter(
        grid, [LIST_ROW + (pos >> 9), pos & (W - 1)], _f(entry), mask=msk)
    return cnt + jnp.sum(mi)
  cnt = lax.fori_loop(0, MGROUPS, compact, jnp.int32(0))

  pad = cnt + iota
  plsc.store_scatter(
      grid, [LIST_ROW + (pad >> 9), pad & (W - 1)],
      _f(jnp.full((L,), -1, jnp.int32)))

  grid[SH_ROW, pl.ds(0, L)] = _f(jnp.zeros((L,), jnp.int32) + cnt)
  pltpu.sync_copy(
      grid.at[pl.ds(SH_ROW, 1), pl.ds(0, L)],
      hdr.at[bic, pl.ds(0, 1), pl.ds(chunk * L, L)])

  def zero_map2(i, _):
    for k in range(4):
      g = i * 4 + k
      grid[g >> 5, pl.ds((g & 31) * L, L)] = fzeros
    return _
  lax.fori_loop(0, MGROUPS // 4, zero_map2, None)

  plsc.subcore_barrier()

  pltpu.sync_copy(hdr.at[bic, pl.ds(0, 1)], grid.at[pl.ds(SH_ROW, 1)])
  cnts = []
  ngroups = []
  loffs = []
  cur = jnp.int32(0)
  for k in range(CHUNKS_PER_BATCH):
    ck = jnp.max(_i(grid[SH_ROW, pl.ds(k * L, L)]))
    cnts.append(ck)
    ngroups.append((ck + (L - 1)) >> 4)
    loffs.append(cur)
    cur = cur + ((ck + L + (W - 1)) >> 9)

  myoff = jnp.int32(0)
  for k in range(CHUNKS_PER_BATCH):
    myoff = jnp.where(chunk == k, loffs[k], myoff)
  rows_me = (cnt + L + (W - 1)) >> 9

  def pub_row(r, _):
    @pl.when(r < rows_me)
    def _go():
      pltpu.async_copy(
          grid.at[pl.ds(LIST_ROW + r, 1)], xls.at[bic, pl.ds(myoff + r, 1)],
          sf)
    return _
  lax.fori_loop(0, LIST_PUB_ROWS, pub_row, None)

  def drain_pub(i, _):
    pltpu.make_async_copy(
        grid.at[pl.ds(LIST_ROW, 1)], xls.at[bic, pl.ds(0, 1)], sf).wait()
    return _
  lax.fori_loop(0, rows_me, drain_pub, None)

  plsc.subcore_barrier()

  pltpu.sync_copy(xls.at[bic], grid.at[pl.ds(LIST_ROW, 32)])

  loff_vec = jnp.zeros((L,), jnp.int32)
  ngrp_vec = jnp.zeros((L,), jnp.int32)
  for k in range(CHUNKS_PER_BATCH):
    loff_vec = jnp.where(iota == k, loffs[k], loff_vec)
    ngrp_vec = jnp.where(iota == k, ngroups[k], ngrp_vec)

  def sel(vec, k):
    return jnp.max(jnp.where(iota == k, vec, 0))

  def out_dst(c, k):
    return out_hbm.at[b, c, pl.ds(k * ROWS_PER_CHUNK, ROWS_PER_CHUNK)]

  def scatter_list(k, xc, srow, value_pass):
    lr = LIST_ROW + sel(loff_vec, k)
    ng = sel(ngrp_vec, k)

    def gather_group(g, _):
      e = _i(grid[lr + (g >> 5), pl.ds((g & 31) * L, L)])
      live = e >= 0
      pv = e & ((1 << PBITS) - 1)
      li = (e >> PBITS) & (CHUNK_PIX - 1)
      if value_pass:
        vals = plsc.load_gather(pool, [xc + pv], mask=live)
      else:
        vals = fzeros
      plsc.store_scatter(
          grid, [srow + (li >> 9), li & (W - 1)], vals, mask=live)
      return _
    lax.fori_loop(0, ng, gather_group, None)

  def do_step(j, k, xc, srow, sem_s):
    c = c0 + j
    t = j * CHUNKS_PER_BATCH + k

    @pl.when(t >= 2)
    def _wait_and_restore():
      pltpu.make_async_copy(
          grid.at[pl.ds(srow, ROWS_PER_CHUNK)], out_dst(c, k), sem_s).wait()
    @pl.when(t >= 2)
    def _restore():
      scatter_list((k + CHUNKS_PER_BATCH - 2) % CHUNKS_PER_BATCH,
                   xc, srow, value_pass=False)

    scatter_list(k, xc, srow, value_pass=True)
    pltpu.async_copy(grid.at[pl.ds(srow, ROWS_PER_CHUNK)], out_dst(c, k), sem_s)

  for j in range(CPB):
    c = c0 + j
    xc = XC0 if j % 2 == 0 else XC1
    sem_x = sx0 if j % 2 == 0 else sx1
    pltpu.make_async_copy(
        x_hbm.at[pl.ds((b * C + c) * P, P)], pool.at[pl.ds(xc, P)],
        sem_x).wait()

    def kk_body(kk, _, j=j, xc=xc):
      for h in range(2):
        k = kk * 2 + h
        srow = STG0_ROW if h == 0 else STG1_ROW
        sem_s = ss0 if h == 0 else ss1
        do_step(j, k, xc, srow, sem_s)
      return _
    lax.fori_loop(0, CHUNKS_PER_BATCH // 2, kk_body, None)

    if j + 2 < CPB:
      pltpu.async_copy(
          x_hbm.at[pl.ds((b * C + c + 2) * P, P)], pool.at[pl.ds(xc, P)],
          sem_x)

  pltpu.make_async_copy(
      grid.at[pl.ds(STG0_ROW, ROWS_PER_CHUNK)],
      out_dst(c0 + CPB - 1, CHUNKS_PER_BATCH - 2), ss0).wait()
  pltpu.make_async_copy(
      grid.at[pl.ds(STG1_ROW, ROWS_PER_CHUNK)],
      out_dst(c0 + CPB - 1, CHUNKS_PER_BATCH - 1), ss1).wait()


_mesh = plsc.VectorSubcoreMesh(
    core_axis_name="c", subcore_axis_name="s", num_cores=NC, num_subcores=NS)

_sc_scatter = pl.kernel(
    _body,
    out_type=jax.ShapeDtypeStruct((B, C, H, W), jnp.float32),
    mesh=_mesh,
    scratch_types=[
        pltpu.VMEM((POOL,), jnp.float32),
        pltpu.VMEM((GRID_ROWS, W), jnp.float32),
        pltpu.VMEM_SHARED((2, 32, W), jnp.float32),
        pltpu.VMEM_SHARED((2, 1, W), jnp.float32),
        pltpu.SemaphoreType.DMA,
        pltpu.SemaphoreType.DMA,
        pltpu.SemaphoreType.DMA,
        pltpu.SemaphoreType.DMA,
        pltpu.SemaphoreType.DMA,
    ],
    compiler_params=pltpu.CompilerParams(needs_layout_passes=False),
)


@jax.jit
def kernel(x, inds):
  ind_t = lax.bitcast_convert_type(
      jnp.transpose(inds, (2, 0, 1)).reshape(-1), jnp.float32)
  return _sc_scatter(x.reshape(-1), ind_t)
